# Initial kernel scaffold; baseline (speedup 1.0000x reference)
import jax
import jax.numpy as jnp
from jax.experimental import pallas as pl


def kernel(x, edge_index, W1, b1, W2, b2, Wl1, bl1, Wl2, bl2, Wl3, bl3):
    raise NotImplementedError("write your pallas kernel here")


# speedup vs baseline: 2.9650x; 2.9650x over previous
import jax
import jax.numpy as jnp
from jax import lax
from jax.experimental import pallas as pl
from jax.experimental.pallas import tpu as pltpu
from jax.experimental.pallas import tpu_sc as plsc

N_SC = 2
N_TILES = 16
N_WORKERS = N_SC * N_TILES
CH = 96
R_ROWS = 16768
N_RANGES = 3
RPT = R_ROWS // N_TILES


def _edge_agg_kernel(n_nodes, kch, n_slices, width, with_tables):
    n_acc = R_ROWS + 16

    mesh = plsc.VectorSubcoreMesh(core_axis_name="c", subcore_axis_name="s")
    out_type = [jax.ShapeDtypeStruct((N_SC, N_RANGES * R_ROWS, width),
                                     jnp.float32)
                for _ in range(n_slices)]
    scratch = [
        pltpu.VMEM((CH,), jnp.int32),
        pltpu.VMEM((CH,), jnp.int32),
        pltpu.VMEM((CH, width), jnp.float32),
        pltpu.VMEM_SHARED((n_acc, width), jnp.float32),
        pltpu.SemaphoreType.DMA,
    ]
    if with_tables:
        scratch += [pltpu.VMEM((CH,), jnp.int32)]

    def body(*refs):
        n_in = (2 + n_slices) if with_tables else 1
        dst_hbm = refs[0]
        if with_tables:
            src_hbm = refs[1]
            tables = refs[2:n_in]
        outs = refs[n_in:n_in + n_slices]
        if with_tables:
            dstbuf, idxbuf, gbuf, acc, sem, srcbuf = refs[n_in + n_slices:]
        else:
            dstbuf, idxbuf, gbuf, acc, sem = refs[n_in + n_slices:]

        c = lax.axis_index("c")
        t = lax.axis_index("s")
        w = c * N_TILES + t
        base = t * RPT

        def fill_gbuf(value):
            v16 = jnp.full((16,), value, jnp.float32)

            def fz(i, _):
                for k in range(width // 16):
                    gbuf[i, pl.ds(16 * k, 16)] = v16
                return 0
            lax.fori_loop(0, CH, fz, 0)

        def zero_stripe():
            nz, rem = divmod(RPT, CH)
            for z in range(nz):
                pltpu.sync_copy(gbuf, acc.at[pl.ds(base + z * CH, CH)])
            if rem:
                pltpu.sync_copy(gbuf.at[pl.ds(0, rem)],
                                acc.at[pl.ds(base + nz * CH, rem)])
            @pl.when(t == 0)
            def _():
                pltpu.sync_copy(gbuf.at[pl.ds(0, 16)],
                                acc.at[pl.ds(R_ROWS, 16)])

        for r in range(N_RANGES):
            lo = r * R_ROWS
            ft, pv = divmod(max(0, n_nodes - lo), RPT)
            assert pv % 8 == 0

            for s in range(n_slices):
                if with_tables:
                    table = tables[s]
                    fill_gbuf(0.0)

                    @pl.when(c == 0)
                    def _():
                        if ft >= N_TILES:
                            pltpu.sync_copy(table.at[pl.ds(lo + base, RPT)],
                                            acc.at[pl.ds(base, RPT)])
                        else:
                            if ft:
                                @pl.when(t < ft)
                                def _():
                                    pltpu.sync_copy(
                                        table.at[pl.ds(lo + base, RPT)],
                                        acc.at[pl.ds(base, RPT)])
                            if pv:
                                @pl.when(t == ft)
                                def _():
                                    pltpu.sync_copy(
                                        table.at[pl.ds(lo + base, pv)],
                                        acc.at[pl.ds(base, pv)])
                        @pl.when(t == 0)
                        def _():
                            pltpu.sync_copy(gbuf.at[pl.ds(0, 16)],
                                            acc.at[pl.ds(R_ROWS, 16)])

                    @pl.when(c != 0)
                    def _():
                        zero_stripe()
                else:
                    fill_gbuf(0.0)
                    zero_stripe()
                plsc.subcore_barrier()
                if not with_tables:
                    fill_gbuf(1.0)

                def step(j, _):
                    pltpu.sync_copy(dst_hbm.at[w, j], dstbuf)
                    for k in range(CH // 16):
                        d = dstbuf[pl.ds(16 * k, 16)]
                        loc = d - lo
                        oob = (loc < 0) | (loc >= R_ROWS)
                        idxbuf[pl.ds(16 * k, 16)] = jnp.where(
                            oob, jnp.full((16,), R_ROWS, jnp.int32), loc)
                    if with_tables:
                        pltpu.sync_copy(src_hbm.at[w, j], srcbuf)
                        pltpu.async_copy(table.at[srcbuf], gbuf, sem).wait()
                    pltpu.sync_copy(gbuf, acc.at[idxbuf], add=True)
                    return 0
                lax.fori_loop(0, kch, step, 0)
                plsc.subcore_barrier()

                pltpu.sync_copy(acc.at[pl.ds(base, RPT)],
                                outs[s].at[c, pl.ds(lo + base, RPT)])
                plsc.subcore_barrier()

    return pl.kernel(
        body, out_type=out_type, mesh=mesh, scratch_types=scratch,
        compiler_params=pltpu.CompilerParams(use_tc_tiling_on_sc=False))


def _t1_body(x_ref, w1_ref, dp_ref, o0, o1, o2, dinv_ref):
    deg = dp_ref[0, :, 0:1] + dp_ref[1, :, 0:1] + 1.0
    dinv = lax.rsqrt(deg)
    h = jnp.dot(x_ref[...], w1_ref[...], preferred_element_type=jnp.float32)
    hs = h * dinv
    for i, o in enumerate((o0, o1, o2)):
        o[...] = hs[:, 32 * i:32 * (i + 1)]
    dinv_ref[...] = dinv


def _t2_body(a0, a1, a2, dinv_ref, b1_ref, w2_ref, o0, o1):
    dinv = dinv_ref[...]
    agg = jnp.concatenate([a[0] + a[1] for a in (a0, a1, a2)], axis=1)
    h1f = jax.nn.relu(dinv * agg + b1_ref[...])
    h2 = jnp.dot(h1f, w2_ref[...], preferred_element_type=jnp.float32)
    hs2 = h2 * dinv
    o0[...] = hs2[:, 0:32]
    o1[...] = hs2[:, 16:48]


def _t3_body(a0, a1, dinv_ref, b2_ref, out_ref):
    dinv = dinv_ref[...]
    agg = jnp.concatenate([a0[0] + a0[1],
                           (a1[0] + a1[1])[:, 16:32]], axis=1)
    out_ref[...] = jax.nn.relu(dinv * agg + b2_ref[...])


def _t4_body(g_ref, wl1_ref, bl1_ref, wl2_ref, bl2_ref, wl3_ref, bl3_ref,
             out_ref):
    g = jax.nn.relu(jnp.dot(g_ref[...], wl1_ref[...],
                            preferred_element_type=jnp.float32) + bl1_ref[...])
    g = jax.nn.relu(jnp.dot(g, wl2_ref[...],
                            preferred_element_type=jnp.float32) + bl2_ref[...])
    g = jnp.dot(g, wl3_ref[...], preferred_element_type=jnp.float32) + bl3_ref[...]
    out_ref[...] = jax.nn.sigmoid(g)


def kernel(x, edge_index, W1, b1, W2, b2, Wl1, bl1, Wl2, bl2, Wl3, bl3):
    n, f_in = x.shape
    f1 = W1.shape[1]
    f2 = W2.shape[1]
    e = edge_index.shape[1]
    kch = -(-e // (N_WORKERS * CH))
    e_pad = N_WORKERS * kch * CH
    r = 1000
    grid = n // r

    ei = edge_index.astype(jnp.int32)
    src = jnp.concatenate([ei[0], jnp.zeros((e_pad - e,), jnp.int32)])
    dst = jnp.concatenate([ei[1], jnp.full((e_pad - e,), n, jnp.int32)])
    src_l = src.reshape(N_WORKERS, kch, CH)
    dst_l = dst.reshape(N_WORKERS, kch, CH)

    deg_fn = _edge_agg_kernel(n, kch, 1, 16, with_tables=False)
    (degp,) = deg_fn(dst_l)

    slice_spec = pl.BlockSpec((r, 32), lambda i: (i, 0))
    part_spec = pl.BlockSpec((N_SC, r, 32), lambda i: (0, i, 0))
    slice_shape = jax.ShapeDtypeStruct((n, 32), jnp.float32)
    t1 = pl.pallas_call(
        _t1_body,
        grid=(grid,),
        in_specs=[
            pl.BlockSpec((r, f_in), lambda i: (i, 0)),
            pl.BlockSpec((f_in, f1), lambda i: (0, 0)),
            pl.BlockSpec((N_SC, r, 16), lambda i: (0, i, 0)),
        ],
        out_specs=[slice_spec] * 3 + [pl.BlockSpec((r, 1), lambda i: (i, 0))],
        out_shape=[slice_shape] * 3 +
                  [jax.ShapeDtypeStruct((n, 1), jnp.float32)],
    )
    *hs1, dinv = t1(x, W1, degp)

    agg1_fn = _edge_agg_kernel(n, kch, 3, 32, with_tables=True)
    a1 = agg1_fn(dst_l, src_l, *hs1)

    t2 = pl.pallas_call(
        _t2_body,
        grid=(grid,),
        in_specs=[part_spec] * 3 + [
            pl.BlockSpec((r, 1), lambda i: (i, 0)),
            pl.BlockSpec((1, f1), lambda i: (0, 0)),
            pl.BlockSpec((f1, f2), lambda i: (0, 0)),
        ],
        out_specs=[slice_spec] * 2,
        out_shape=[slice_shape] * 2,
    )
    hs2 = t2(*a1, dinv, b1.reshape(1, f1), W2)

    agg2_fn = _edge_agg_kernel(n, kch, 2, 32, with_tables=True)
    a2 = agg2_fn(dst_l, src_l, *hs2)

    t3 = pl.pallas_call(
        _t3_body,
        grid=(grid,),
        in_specs=[part_spec] * 2 + [
            pl.BlockSpec((r, 1), lambda i: (i, 0)),
            pl.BlockSpec((1, f2), lambda i: (0, 0)),
        ],
        out_specs=pl.BlockSpec((r, f2), lambda i: (i, 0)),
        out_shape=jax.ShapeDtypeStruct((n, f2), jnp.float32),
    )
    h2f = t3(*a2, dinv, b2.reshape(1, f2))

    groups = n // 40
    g = h2f.reshape(groups, 40 * f2)
    t4 = pl.pallas_call(
        _t4_body,
        out_shape=jax.ShapeDtypeStruct((groups, 1), jnp.float32),
    )
    return t4(g, Wl1, bl1.reshape(1, -1), Wl2, bl2.reshape(1, -1),
              Wl3, bl3.reshape(1, -1))


# speedup vs baseline: 3.9152x; 1.3205x over previous
import jax
import jax.numpy as jnp
from jax import lax
from jax.experimental import pallas as pl
from jax.experimental.pallas import tpu as pltpu
from jax.experimental.pallas import tpu_sc as plsc

N_SC = 2
N_TILES = 16
N_WORKERS = N_SC * N_TILES
CH = 96
R_ROWS = 16768
N_RANGES = 3
RPT = R_ROWS // N_TILES


def _edge_agg_kernel(n_nodes, kch, n_slices, width, with_tables):
    n_acc = R_ROWS + 16

    mesh = plsc.VectorSubcoreMesh(core_axis_name="c", subcore_axis_name="s")
    out_type = [jax.ShapeDtypeStruct((N_SC, N_RANGES * R_ROWS, width),
                                     jnp.float32)
                for _ in range(n_slices)]
    scratch = [
        pltpu.VMEM((CH,), jnp.int32),
        pltpu.VMEM((CH,), jnp.int32),
        pltpu.VMEM((CH,), jnp.int32),
        pltpu.VMEM((CH,), jnp.int32),
        pltpu.VMEM((CH, width), jnp.float32),
        pltpu.VMEM((CH, width), jnp.float32),
        pltpu.VMEM_SHARED((n_acc, width), jnp.float32),
        pltpu.SemaphoreType.DMA,
        pltpu.SemaphoreType.DMA,
        pltpu.SemaphoreType.DMA,
        pltpu.SemaphoreType.DMA,
        pltpu.SemaphoreType.DMA,
        pltpu.SemaphoreType.DMA,
    ]
    if with_tables:
        scratch += [
            pltpu.VMEM((CH,), jnp.int32),
            pltpu.VMEM((CH,), jnp.int32),
            pltpu.SemaphoreType.DMA,
            pltpu.SemaphoreType.DMA,
        ]

    def body(*refs):
        n_in = (2 + n_slices) if with_tables else 1
        dst_hbm = refs[0]
        if with_tables:
            src_hbm = refs[1]
            tables = refs[2:n_in]
        outs = refs[n_in:n_in + n_slices]
        sc = refs[n_in + n_slices:]
        dstbuf = sc[0:2]
        idxbuf = sc[2:4]
        gbufs = sc[4:6]
        acc = sc[6]
        ldsem = sc[7:9]
        gsem = sc[9:11]
        ssem = sc[11:13]
        if with_tables:
            srcbuf = sc[13:15]
            lssem = sc[15:17]
        gbuf = gbufs[0]

        c = lax.axis_index("c")
        t = lax.axis_index("s")
        w = c * N_TILES + t
        base = t * RPT

        def fill_gbuf(value):
            v16 = jnp.full((16,), value, jnp.float32)

            def fz(i, _):
                for k in range(width // 16):
                    gbuf[i, pl.ds(16 * k, 16)] = v16
                return 0
            lax.fori_loop(0, CH, fz, 0)

        def zero_stripe():
            nz, rem = divmod(RPT, CH)
            for z in range(nz):
                pltpu.sync_copy(gbuf, acc.at[pl.ds(base + z * CH, CH)])
            if rem:
                pltpu.sync_copy(gbuf.at[pl.ds(0, rem)],
                                acc.at[pl.ds(base + nz * CH, rem)])
            @pl.when(t == 0)
            def _():
                pltpu.sync_copy(gbuf.at[pl.ds(0, 16)],
                                acc.at[pl.ds(R_ROWS, 16)])

        for r in range(N_RANGES):
            lo = r * R_ROWS
            ft, pv = divmod(max(0, n_nodes - lo), RPT)
            assert pv % 8 == 0

            for s in range(n_slices):
                if with_tables:
                    table = tables[s]
                    fill_gbuf(0.0)

                    @pl.when(c == 0)
                    def _():
                        if ft >= N_TILES:
                            pltpu.sync_copy(table.at[pl.ds(lo + base, RPT)],
                                            acc.at[pl.ds(base, RPT)])
                        else:
                            if ft:
                                @pl.when(t < ft)
                                def _():
                                    pltpu.sync_copy(
                                        table.at[pl.ds(lo + base, RPT)],
                                        acc.at[pl.ds(base, RPT)])
                            if pv:
                                @pl.when(t == ft)
                                def _():
                                    pltpu.sync_copy(
                                        table.at[pl.ds(lo + base, pv)],
                                        acc.at[pl.ds(base, pv)])
                        @pl.when(t == 0)
                        def _():
                            pltpu.sync_copy(gbuf.at[pl.ds(0, 16)],
                                            acc.at[pl.ds(R_ROWS, 16)])

                    @pl.when(c != 0)
                    def _():
                        zero_stripe()
                else:
                    fill_gbuf(0.0)
                    zero_stripe()
                plsc.subcore_barrier()
                if not with_tables:
                    fill_gbuf(1.0)

                def load_dst(j, p):
                    pltpu.async_copy(dst_hbm.at[w, j], dstbuf[p], ldsem[p])

                def wait_ld(p):
                    pltpu.make_async_copy(dst_hbm.at[w, 0], dstbuf[p],
                                          ldsem[p]).wait()

                def clamp(p):
                    for k in range(CH // 16):
                        d = dstbuf[p][pl.ds(16 * k, 16)]
                        loc = d - lo
                        oob = (loc < 0) | (loc >= R_ROWS)
                        idxbuf[p][pl.ds(16 * k, 16)] = jnp.where(
                            oob, jnp.full((16,), R_ROWS, jnp.int32), loc)

                if with_tables:
                    tb = table

                    def load_src(j, p):
                        pltpu.async_copy(src_hbm.at[w, j], srcbuf[p],
                                         lssem[p])

                    def wait_ls(p):
                        pltpu.make_async_copy(src_hbm.at[w, 0], srcbuf[p],
                                              lssem[p]).wait()

                    def wait_g(p):
                        pltpu.make_async_copy(tb.at[srcbuf[p]], gbufs[p],
                                              gsem[p]).wait()

                    def issue_s(p):
                        pltpu.async_copy(gbufs[p], acc.at[idxbuf[p]],
                                         ssem[p], add=True)

                    def wait_s(p):
                        pltpu.make_async_copy(gbufs[p], acc.at[idxbuf[p]],
                                              ssem[p]).wait()

                    load_dst(0, 0)
                    load_src(0, 0)

                    def pair(jj, _):
                        for h in (0, 1):
                            p, q = h, 1 - h
                            j = 2 * jj + h
                            wait_ld(p)
                            wait_ls(p)
                            @pl.when(jj >= 1)
                            def _():
                                wait_s(p)
                            clamp(p)
                            pltpu.async_copy(tb.at[srcbuf[p]], gbufs[p],
                                             gsem[p])
                            if h == 0:
                                @pl.when(jj >= 1)
                                def _():
                                    wait_g(q)
                                    issue_s(q)
                                load_dst(j + 1, q)
                                load_src(j + 1, q)
                            else:
                                wait_g(q)
                                issue_s(q)
                                @pl.when(jj < kch // 2 - 1)
                                def _():
                                    load_dst(j + 1, q)
                                    load_src(j + 1, q)
                        return 0
                    lax.fori_loop(0, kch // 2, pair, 0)
                    wait_s(0)
                    wait_g(1)
                    issue_s(1)
                    wait_s(1)
                else:
                    def issue_s(p):
                        pltpu.async_copy(gbuf, acc.at[idxbuf[p]],
                                         ssem[p], add=True)

                    def wait_s(p):
                        pltpu.make_async_copy(gbuf, acc.at[idxbuf[p]],
                                              ssem[p]).wait()

                    load_dst(0, 0)

                    def pair(jj, _):
                        for h in (0, 1):
                            p, q = h, 1 - h
                            j = 2 * jj + h
                            wait_ld(p)
                            @pl.when(jj >= 1)
                            def _():
                                wait_s(p)
                            clamp(p)
                            issue_s(p)
                            if h == 0:
                                load_dst(j + 1, q)
                            else:
                                @pl.when(jj < kch // 2 - 1)
                                def _():
                                    load_dst(j + 1, q)
                        return 0
                    lax.fori_loop(0, kch // 2, pair, 0)
                    wait_s(0)
                    wait_s(1)
                plsc.subcore_barrier()

                pltpu.sync_copy(acc.at[pl.ds(base, RPT)],
                                outs[s].at[c, pl.ds(lo + base, RPT)])
                plsc.subcore_barrier()

    return pl.kernel(
        body, out_type=out_type, mesh=mesh, scratch_types=scratch,
        compiler_params=pltpu.CompilerParams(use_tc_tiling_on_sc=False))


def _t1_body(x_ref, w1_ref, dp_ref, o0, o1, o2, dinv_ref):
    deg = dp_ref[0, :, 0:1] + dp_ref[1, :, 0:1] + 1.0
    dinv = lax.rsqrt(deg)
    h = jnp.dot(x_ref[...], w1_ref[...], preferred_element_type=jnp.float32)
    hs = h * dinv
    for i, o in enumerate((o0, o1, o2)):
        o[...] = hs[:, 32 * i:32 * (i + 1)]
    dinv_ref[...] = dinv


def _t2_body(a0, a1, a2, dinv_ref, b1_ref, w2_ref, o0, o1):
    dinv = dinv_ref[...]
    agg = jnp.concatenate([a[0] + a[1] for a in (a0, a1, a2)], axis=1)
    h1f = jax.nn.relu(dinv * agg + b1_ref[...])
    h2 = jnp.dot(h1f, w2_ref[...], preferred_element_type=jnp.float32)
    hs2 = h2 * dinv
    o0[...] = hs2[:, 0:32]
    o1[...] = hs2[:, 16:48]


def _t3_body(a0, a1, dinv_ref, b2_ref, out_ref):
    dinv = dinv_ref[...]
    agg = jnp.concatenate([a0[0] + a0[1],
                           (a1[0] + a1[1])[:, 16:32]], axis=1)
    out_ref[...] = jax.nn.relu(dinv * agg + b2_ref[...])


def _t4_body(g_ref, wl1_ref, bl1_ref, wl2_ref, bl2_ref, wl3_ref, bl3_ref,
             out_ref):
    g = jax.nn.relu(jnp.dot(g_ref[...], wl1_ref[...],
                            preferred_element_type=jnp.float32) + bl1_ref[...])
    g = jax.nn.relu(jnp.dot(g, wl2_ref[...],
                            preferred_element_type=jnp.float32) + bl2_ref[...])
    g = jnp.dot(g, wl3_ref[...], preferred_element_type=jnp.float32) + bl3_ref[...]
    out_ref[...] = jax.nn.sigmoid(g)


def kernel(x, edge_index, W1, b1, W2, b2, Wl1, bl1, Wl2, bl2, Wl3, bl3):
    n, f_in = x.shape
    f1 = W1.shape[1]
    f2 = W2.shape[1]
    e = edge_index.shape[1]
    kch = -(-e // (N_WORKERS * CH))
    kch += kch % 2
    e_pad = N_WORKERS * kch * CH
    r = 1000
    grid = n // r

    ei = edge_index.astype(jnp.int32)
    src = jnp.concatenate([ei[0], jnp.zeros((e_pad - e,), jnp.int32)])
    dst = jnp.concatenate([ei[1], jnp.full((e_pad - e,), n, jnp.int32)])
    src_l = src.reshape(N_WORKERS, kch, CH)
    dst_l = dst.reshape(N_WORKERS, kch, CH)

    deg_fn = _edge_agg_kernel(n, kch, 1, 16, with_tables=False)
    (degp,) = deg_fn(dst_l)

    slice_spec = pl.BlockSpec((r, 32), lambda i: (i, 0))
    part_spec = pl.BlockSpec((N_SC, r, 32), lambda i: (0, i, 0))
    slice_shape = jax.ShapeDtypeStruct((n, 32), jnp.float32)
    t1 = pl.pallas_call(
        _t1_body,
        grid=(grid,),
        in_specs=[
            pl.BlockSpec((r, f_in), lambda i: (i, 0)),
            pl.BlockSpec((f_in, f1), lambda i: (0, 0)),
            pl.BlockSpec((N_SC, r, 16), lambda i: (0, i, 0)),
        ],
        out_specs=[slice_spec] * 3 + [pl.BlockSpec((r, 1), lambda i: (i, 0))],
        out_shape=[slice_shape] * 3 +
                  [jax.ShapeDtypeStruct((n, 1), jnp.float32)],
    )
    *hs1, dinv = t1(x, W1, degp)

    agg1_fn = _edge_agg_kernel(n, kch, 3, 32, with_tables=True)
    a1 = agg1_fn(dst_l, src_l, *hs1)

    t2 = pl.pallas_call(
        _t2_body,
        grid=(grid,),
        in_specs=[part_spec] * 3 + [
            pl.BlockSpec((r, 1), lambda i: (i, 0)),
            pl.BlockSpec((1, f1), lambda i: (0, 0)),
            pl.BlockSpec((f1, f2), lambda i: (0, 0)),
        ],
        out_specs=[slice_spec] * 2,
        out_shape=[slice_shape] * 2,
    )
    hs2 = t2(*a1, dinv, b1.reshape(1, f1), W2)

    agg2_fn = _edge_agg_kernel(n, kch, 2, 32, with_tables=True)
    a2 = agg2_fn(dst_l, src_l, *hs2)

    t3 = pl.pallas_call(
        _t3_body,
        grid=(grid,),
        in_specs=[part_spec] * 2 + [
            pl.BlockSpec((r, 1), lambda i: (i, 0)),
            pl.BlockSpec((1, f2), lambda i: (0, 0)),
        ],
        out_specs=pl.BlockSpec((r, f2), lambda i: (i, 0)),
        out_shape=jax.ShapeDtypeStruct((n, f2), jnp.float32),
    )
    h2f = t3(*a2, dinv, b2.reshape(1, f2))

    groups = n // 40
    g = h2f.reshape(groups, 40 * f2)
    t4 = pl.pallas_call(
        _t4_body,
        out_shape=jax.ShapeDtypeStruct((groups, 1), jnp.float32),
    )
    return t4(g, Wl1, bl1.reshape(1, -1), Wl2, bl2.reshape(1, -1),
              Wl3, bl3.reshape(1, -1))


# speedup vs baseline: 4.7237x; 1.2065x over previous
import jax
import jax.numpy as jnp
from jax import lax
from jax.experimental import pallas as pl
from jax.experimental.pallas import tpu as pltpu
from jax.experimental.pallas import tpu_sc as plsc

N_SC = 2
N_TILES = 16
N_WORKERS = N_SC * N_TILES
CH = 96
R_ROWS = 16768
N_RANGES = 3
RPT = R_ROWS // N_TILES
KCAP = 96
CAP = KCAP * CH
LBUF = CAP + 128

_SC_PARAMS = pltpu.CompilerParams(use_tc_tiling_on_sc=False)


def _partition_kernel(kch):
    mesh = plsc.VectorSubcoreMesh(core_axis_name="c", subcore_axis_name="s")
    out_type = [jax.ShapeDtypeStruct((N_WORKERS, N_RANGES, CAP), jnp.int32)
                for _ in range(2)]
    scratch = (
        [pltpu.VMEM((CH,), jnp.int32) for _ in range(4)]
        + [pltpu.VMEM((LBUF,), jnp.int32) for _ in range(6)]
        + [pltpu.SemaphoreType.DMA for _ in range(4)]
    )

    def body(dst_hbm, src_hbm, pdst_out, psrc_out, *sc):
        dstbuf = sc[0:2]
        srcbuf = sc[2:4]
        dlists = sc[4:7]
        slists = sc[7:10]
        ldsem = sc[10:12]
        lssem = sc[12:14]

        c = lax.axis_index("c")
        t = lax.axis_index("s")
        w = c * N_TILES + t

        dump16 = jnp.full((16,), R_ROWS, jnp.int32)
        zero16 = jnp.zeros((16,), jnp.int32)

        for r in range(N_RANGES):
            dl, sl = dlists[r], slists[r]

            def fill(i, _):
                dl[pl.ds(16 * i, 16)] = dump16
                sl[pl.ds(16 * i, 16)] = zero16
                return 0
            lax.fori_loop(0, LBUF // 16, fill, 0)

        def load(j, p):
            pltpu.async_copy(dst_hbm.at[w, j], dstbuf[p], ldsem[p])
            pltpu.async_copy(src_hbm.at[w, j], srcbuf[p], lssem[p])

        def wait_load(p):
            pltpu.make_async_copy(dst_hbm.at[w, 0], dstbuf[p],
                                  ldsem[p]).wait()
            pltpu.make_async_copy(src_hbm.at[w, 0], srcbuf[p],
                                  lssem[p]).wait()

        load(0, 0)

        def pair(jj, offs):
            for h in (0, 1):
                p, q = h, 1 - h
                j = 2 * jj + h
                wait_load(p)
                if h == 0:
                    load(j + 1, q)
                else:
                    @pl.when(jj < kch // 2 - 1)
                    def _():
                        load(j + 1, q)
                trash = jnp.full((16,), LBUF - 16, jnp.int32)
                lane = lax.iota(jnp.int32, 16)
                for k in range(CH // 16):
                    dv = dstbuf[p][pl.ds(16 * k, 16)]
                    sv = srcbuf[p][pl.ds(16 * k, 16)]
                    masks = []
                    packed = jnp.zeros((16,), jnp.int32)
                    for r in range(N_RANGES):
                        loc = dv - r * R_ROWS
                        m = (loc >= 0) & (loc < R_ROWS)
                        masks.append(m)
                        packed = packed + (m.astype(jnp.int32) << (8 * r))
                    cs = packed
                    for sh in (1, 2, 4, 8):
                        moved = lax.gather(
                            cs, jnp.maximum(lane - sh, 0)[:, None],
                            lax.GatherDimensionNumbers(
                                offset_dims=(), collapsed_slice_dims=(0,),
                                start_index_map=(0,)),
                            (1,),
                            mode=lax.GatherScatterMode.PROMISE_IN_BOUNDS)
                        cs = cs + jnp.where(lane >= sh, moved, 0)
                    new = []
                    for r in range(N_RANGES):
                        m = masks[r]
                        mi = m.astype(jnp.int32)
                        csr = (cs >> (8 * r)) & 0xFF
                        slot = jnp.where(m, offs[r] + (csr - mi), trash)
                        plsc.store_scatter(dlists[r], [slot], dv - r * R_ROWS)
                        plsc.store_scatter(slists[r], [slot], sv)
                        new.append(offs[r] + csr[15])
                    offs = tuple(new)
            return offs
        lax.fori_loop(0, kch // 2, pair,
                      (jnp.int32(0), jnp.int32(0), jnp.int32(0)))

        for r in range(N_RANGES):
            pltpu.sync_copy(dlists[r].at[pl.ds(0, CAP)], pdst_out.at[w, r])
            pltpu.sync_copy(slists[r].at[pl.ds(0, CAP)], psrc_out.at[w, r])

    return pl.kernel(
        body, out_type=out_type, mesh=mesh, scratch_types=scratch,
        compiler_params=pltpu.CompilerParams(use_tc_tiling_on_sc=False,
                                             needs_layout_passes=False))


def _edge_agg_kernel(n_nodes, n_slices, width, with_tables):
    n_acc = R_ROWS + 16

    mesh = plsc.VectorSubcoreMesh(core_axis_name="c", subcore_axis_name="s")
    out_type = [jax.ShapeDtypeStruct((N_SC, N_RANGES * R_ROWS, width),
                                     jnp.float32)
                for _ in range(n_slices)]
    scratch = (
        [pltpu.VMEM((CH,), jnp.int32) for _ in range(4)]
        + [pltpu.VMEM((CH, width), jnp.float32) for _ in range(2)]
        + [pltpu.VMEM_SHARED((n_acc, width), jnp.float32)]
        + [pltpu.SemaphoreType.DMA for _ in range(6)]
    )
    if with_tables:
        scratch += (
            [pltpu.VMEM((CH,), jnp.int32) for _ in range(2)]
            + [pltpu.SemaphoreType.DMA for _ in range(4)]
        )

    def body(*refs):
        n_in = (2 + n_slices) if with_tables else 1
        pdst = refs[0]
        if with_tables:
            psrc = refs[1]
            tables = refs[2:n_in]
        outs = refs[n_in:n_in + n_slices]
        sc = refs[n_in + n_slices:]
        dstbuf = sc[0:4]
        gbufs = sc[4:6]
        acc = sc[6]
        ldsem = sc[7:11]
        ssem = sc[11:13]
        if with_tables:
            srcbuf = sc[13:15]
            lssem = sc[15:17]
            gsem = sc[17:19]
        gbuf = gbufs[0]

        c = lax.axis_index("c")
        t = lax.axis_index("s")
        w = c * N_TILES + t
        base = t * RPT

        def fill_gbuf(value):
            v16 = jnp.full((16,), value, jnp.float32)

            def fz(i, _):
                for k in range(width // 16):
                    gbuf[i, pl.ds(16 * k, 16)] = v16
                return 0
            lax.fori_loop(0, CH, fz, 0)

        def zero_stripe():
            nz, rem = divmod(RPT, CH)
            for z in range(nz):
                pltpu.sync_copy(gbuf, acc.at[pl.ds(base + z * CH, CH)])
            if rem:
                pltpu.sync_copy(gbuf.at[pl.ds(0, rem)],
                                acc.at[pl.ds(base + nz * CH, rem)])
            @pl.when(t == 0)
            def _():
                pltpu.sync_copy(gbuf.at[pl.ds(0, 16)],
                                acc.at[pl.ds(R_ROWS, 16)])

        def load_dst(r, j, d):
            pltpu.async_copy(pdst.at[w, r, pl.ds(j * CH, CH)], dstbuf[d],
                             ldsem[d])

        def wait_ld(d):
            pltpu.make_async_copy(pdst.at[w, 0, pl.ds(0, CH)], dstbuf[d],
                                  ldsem[d]).wait()

        for r in range(N_RANGES):
            lo = r * R_ROWS
            ft, pv = divmod(max(0, n_nodes - lo), RPT)
            assert pv % 8 == 0

            for s in range(n_slices):
                if with_tables:
                    table = tables[s]
                    fill_gbuf(0.0)

                    @pl.when(c == 0)
                    def _():
                        if ft >= N_TILES:
                            pltpu.sync_copy(table.at[pl.ds(lo + base, RPT)],
                                            acc.at[pl.ds(base, RPT)])
                        else:
                            if ft:
                                @pl.when(t < ft)
                                def _():
                                    pltpu.sync_copy(
                                        table.at[pl.ds(lo + base, RPT)],
                                        acc.at[pl.ds(base, RPT)])
                            if pv:
                                @pl.when(t == ft)
                                def _():
                                    pltpu.sync_copy(
                                        table.at[pl.ds(lo + base, pv)],
                                        acc.at[pl.ds(base, pv)])
                        @pl.when(t == 0)
                        def _():
                            pltpu.sync_copy(gbuf.at[pl.ds(0, 16)],
                                            acc.at[pl.ds(R_ROWS, 16)])

                    @pl.when(c != 0)
                    def _():
                        zero_stripe()
                else:
                    fill_gbuf(0.0)
                    zero_stripe()
                plsc.subcore_barrier()
                if not with_tables:
                    fill_gbuf(1.0)

                if with_tables:
                    tb = table

                    def load_src(j, p):
                        pltpu.async_copy(psrc.at[w, r, pl.ds(j * CH, CH)],
                                         srcbuf[p], lssem[p])

                    def wait_ls(p):
                        pltpu.make_async_copy(psrc.at[w, 0, pl.ds(0, CH)],
                                              srcbuf[p], lssem[p]).wait()

                    def wait_g(p):
                        pltpu.make_async_copy(tb.at[srcbuf[p]], gbufs[p],
                                              gsem[p]).wait()

                    def issue_s(p, d):
                        pltpu.async_copy(gbufs[p], acc.at[dstbuf[d]],
                                         ssem[p], add=True)

                    def wait_s(p):
                        pltpu.make_async_copy(gbufs[p], acc.at[dstbuf[0]],
                                              ssem[p]).wait()

                    load_dst(r, 0, 0)
                    load_src(0, 0)

                    def quad(jj, _):
                        for h in (0, 1, 2, 3):
                            p, q = h % 2, 1 - h % 2
                            j = 4 * jj + h
                            wait_ld(h)
                            wait_ls(p)
                            if h < 2:
                                @pl.when(jj >= 1)
                                def _():
                                    wait_s(p)
                            else:
                                wait_s(p)
                            pltpu.async_copy(tb.at[srcbuf[p]], gbufs[p],
                                             gsem[p])
                            if h == 0:
                                @pl.when(jj >= 1)
                                def _():
                                    wait_g(q)
                                    issue_s(q, 3)
                            else:
                                wait_g(q)
                                issue_s(q, h - 1)
                            if h == 3:
                                @pl.when(jj < KCAP // 4 - 1)
                                def _():
                                    load_dst(r, j + 1, 0)
                                    load_src(j + 1, q)
                            else:
                                load_dst(r, j + 1, h + 1)
                                load_src(j + 1, q)
                        return 0
                    lax.fori_loop(0, KCAP // 4, quad, 0)
                    wait_s(0)
                    wait_g(1)
                    issue_s(1, 3)
                    wait_s(1)
                else:
                    def issue_s(p, d):
                        pltpu.async_copy(gbuf, acc.at[dstbuf[d]],
                                         ssem[p], add=True)

                    def wait_s(p):
                        pltpu.make_async_copy(gbuf, acc.at[dstbuf[0]],
                                              ssem[p]).wait()

                    load_dst(r, 0, 0)

                    def quad(jj, _):
                        for h in (0, 1, 2, 3):
                            p = h % 2
                            j = 4 * jj + h
                            wait_ld(h)
                            if h < 2:
                                @pl.when(jj >= 1)
                                def _():
                                    wait_s(p)
                            else:
                                wait_s(p)
                            issue_s(p, h)
                            if h == 3:
                                @pl.when(jj < KCAP // 4 - 1)
                                def _():
                                    load_dst(r, j + 1, 0)
                            else:
                                load_dst(r, j + 1, h + 1)
                        return 0
                    lax.fori_loop(0, KCAP // 4, quad, 0)
                    wait_s(0)
                    wait_s(1)
                plsc.subcore_barrier()

                pltpu.sync_copy(acc.at[pl.ds(base, RPT)],
                                outs[s].at[c, pl.ds(lo + base, RPT)])
                plsc.subcore_barrier()

    return pl.kernel(body, out_type=out_type, mesh=mesh,
                     scratch_types=scratch, compiler_params=_SC_PARAMS)


def _t1_body(x_ref, w1_ref, dp_ref, o0, o1, o2, dinv_ref):
    deg = dp_ref[0, :, 0:1] + dp_ref[1, :, 0:1] + 1.0
    dinv = lax.rsqrt(deg)
    h = jnp.dot(x_ref[...], w1_ref[...], preferred_element_type=jnp.float32)
    hs = h * dinv
    for i, o in enumerate((o0, o1, o2)):
        o[...] = hs[:, 32 * i:32 * (i + 1)]
    dinv_ref[...] = dinv


def _t2_body(a0, a1, a2, dinv_ref, b1_ref, w2_ref, o0, o1):
    dinv = dinv_ref[...]
    agg = jnp.concatenate([a[0] + a[1] for a in (a0, a1, a2)], axis=1)
    h1f = jax.nn.relu(dinv * agg + b1_ref[...])
    h2 = jnp.dot(h1f, w2_ref[...], preferred_element_type=jnp.float32)
    hs2 = h2 * dinv
    o0[...] = hs2[:, 0:32]
    o1[...] = hs2[:, 16:48]


def _t3_body(a0, a1, dinv_ref, b2_ref, out_ref):
    dinv = dinv_ref[...]
    agg = jnp.concatenate([a0[0] + a0[1],
                           (a1[0] + a1[1])[:, 16:32]], axis=1)
    out_ref[...] = jax.nn.relu(dinv * agg + b2_ref[...])


def _t4_body(g_ref, wl1_ref, bl1_ref, wl2_ref, bl2_ref, wl3_ref, bl3_ref,
             out_ref):
    g = jax.nn.relu(jnp.dot(g_ref[...], wl1_ref[...],
                            preferred_element_type=jnp.float32) + bl1_ref[...])
    g = jax.nn.relu(jnp.dot(g, wl2_ref[...],
                            preferred_element_type=jnp.float32) + bl2_ref[...])
    g = jnp.dot(g, wl3_ref[...], preferred_element_type=jnp.float32) + bl3_ref[...]
    out_ref[...] = jax.nn.sigmoid(g)


def kernel(x, edge_index, W1, b1, W2, b2, Wl1, bl1, Wl2, bl2, Wl3, bl3):
    n, f_in = x.shape
    f1 = W1.shape[1]
    f2 = W2.shape[1]
    e = edge_index.shape[1]
    kch = -(-e // (N_WORKERS * CH))
    kch += kch % 2
    e_pad = N_WORKERS * kch * CH
    r = 1000
    grid = n // r

    ei = edge_index.astype(jnp.int32)
    src = jnp.concatenate([ei[0], jnp.zeros((e_pad - e,), jnp.int32)])
    dst = jnp.concatenate([ei[1], jnp.full((e_pad - e,), n, jnp.int32)])
    src_l = src.reshape(N_WORKERS, kch, CH)
    dst_l = dst.reshape(N_WORKERS, kch, CH)

    pdst, psrc = _partition_kernel(kch)(dst_l, src_l)

    deg_fn = _edge_agg_kernel(n, 1, 16, with_tables=False)
    (degp,) = deg_fn(pdst)

    slice_spec = pl.BlockSpec((r, 32), lambda i: (i, 0))
    part_spec = pl.BlockSpec((N_SC, r, 32), lambda i: (0, i, 0))
    slice_shape = jax.ShapeDtypeStruct((n, 32), jnp.float32)
    t1 = pl.pallas_call(
        _t1_body,
        grid=(grid,),
        in_specs=[
            pl.BlockSpec((r, f_in), lambda i: (i, 0)),
            pl.BlockSpec((f_in, f1), lambda i: (0, 0)),
            pl.BlockSpec((N_SC, r, 16), lambda i: (0, i, 0)),
        ],
        out_specs=[slice_spec] * 3 + [pl.BlockSpec((r, 1), lambda i: (i, 0))],
        out_shape=[slice_shape] * 3 +
                  [jax.ShapeDtypeStruct((n, 1), jnp.float32)],
    )
    *hs1, dinv = t1(x, W1, degp)

    agg1_fn = _edge_agg_kernel(n, 3, 32, with_tables=True)
    a1 = agg1_fn(pdst, psrc, *hs1)

    t2 = pl.pallas_call(
        _t2_body,
        grid=(grid,),
        in_specs=[part_spec] * 3 + [
            pl.BlockSpec((r, 1), lambda i: (i, 0)),
            pl.BlockSpec((1, f1), lambda i: (0, 0)),
            pl.BlockSpec((f1, f2), lambda i: (0, 0)),
        ],
        out_specs=[slice_spec] * 2,
        out_shape=[slice_shape] * 2,
    )
    hs2 = t2(*a1, dinv, b1.reshape(1, f1), W2)

    agg2_fn = _edge_agg_kernel(n, 2, 32, with_tables=True)
    a2 = agg2_fn(pdst, psrc, *hs2)

    t3 = pl.pallas_call(
        _t3_body,
        grid=(grid,),
        in_specs=[part_spec] * 2 + [
            pl.BlockSpec((r, 1), lambda i: (i, 0)),
            pl.BlockSpec((1, f2), lambda i: (0, 0)),
        ],
        out_specs=pl.BlockSpec((r, f2), lambda i: (i, 0)),
        out_shape=jax.ShapeDtypeStruct((n, f2), jnp.float32),
    )
    h2f = t3(*a2, dinv, b2.reshape(1, f2))

    groups = n // 40
    g = h2f.reshape(groups, 40 * f2)
    t4 = pl.pallas_call(
        _t4_body,
        out_shape=jax.ShapeDtypeStruct((groups, 1), jnp.float32),
    )
    return t4(g, Wl1, bl1.reshape(1, -1), Wl2, bl2.reshape(1, -1),
              Wl3, bl3.reshape(1, -1))
